# baseline (speedup 1.0000x reference)
import jax
import jax.numpy as jnp
from jax import lax
from jax.experimental import pallas as pl

_F32 = jnp.float32
_HI = lax.Precision.DEFAULT


def _fused_kernel(feat_ref, adj_ref, mask_ref,
                  e1w_ref, e1b_ref, e2w_ref, e2b_ref,
                  sgw_ref, sgb_ref, gdw_ref, gdb_ref,
                  p1wg_ref, p1wx_ref, p1b_ref, p2w_ref, p2b_ref,
                  vw_ref, vb_ref, out_ref):
    x = jnp.maximum(
        jnp.dot(feat_ref[...], e1w_ref[...], preferred_element_type=_F32,
                precision=_HI) + e1b_ref[...], 0.0)
    x = jnp.maximum(
        jnp.dot(x, e2w_ref[...], preferred_element_type=_F32,
                precision=_HI) + e2b_ref[...], 0.0)

    adj = adj_ref[...]
    deg = jnp.sum(adj, axis=0, keepdims=True)
    d_row = jnp.where(deg > 0.0, lax.rsqrt(deg), 0.0)
    d_col = d_row.reshape(adj.shape[0], 1)

    contract_rows = (((0,), (0,)), ((), ()))
    adj16 = adj.astype(jnp.bfloat16)
    h = x
    for _ in range(2):
        y = (d_col * h).astype(jnp.bfloat16)
        t = lax.dot_general(adj16, y, contract_rows,
                            preferred_element_type=_F32)
        h = d_col * t

    h = jnp.maximum(
        jnp.dot(h, sgw_ref[...], preferred_element_type=_F32,
                precision=_HI) + sgb_ref[...], 0.0)
    x_graph = jnp.maximum(
        jnp.dot(h, gdw_ref[...], preferred_element_type=_F32,
                precision=_HI) + gdb_ref[...], 0.0)

    p = jnp.maximum(
        jnp.dot(x_graph, p1wg_ref[...], preferred_element_type=_F32,
                precision=_HI)
        + jnp.dot(x, p1wx_ref[...], preferred_element_type=_F32,
                  precision=_HI)
        + p1b_ref[...], 0.0)
    p = jnp.maximum(
        jnp.dot(p, p2w_ref[...], preferred_element_type=_F32,
                precision=_HI) + p2b_ref[...], 0.0)
    value = jnp.dot(p, vw_ref[...], preferred_element_type=_F32,
                    precision=_HI) + vb_ref[...]
    out_ref[...] = value * mask_ref[...]


def kernel(features, adjacency, mask, enc1_w, enc1_b, enc2_w, enc2_b,
           sg_w, sg_b, gd_w, gd_b, p1_w, p1_b, p2_w, p2_b, v_w, v_b):
    n = features.shape[0]
    f_graph = sg_w.shape[1]
    args = (
        features, adjacency, mask.reshape(n, 1),
        enc1_w, enc1_b.reshape(1, -1), enc2_w, enc2_b.reshape(1, -1),
        sg_w, sg_b.reshape(1, -1), gd_w, gd_b.reshape(1, -1),
        p1_w[:f_graph], p1_w[f_graph:], p1_b.reshape(1, -1),
        p2_w, p2_b.reshape(1, -1), v_w, v_b.reshape(1, -1),
    )
    return pl.pallas_call(
        _fused_kernel,
        out_shape=jax.ShapeDtypeStruct((n, 1), jnp.float32),
    )(*args)


# speedup vs baseline: 1.0701x; 1.0701x over previous
import jax
import jax.numpy as jnp
from jax import lax
from jax.experimental import pallas as pl
from jax.experimental.pallas import tpu as pltpu

_F32 = jnp.float32
_N_CHUNKS = 8


def _fused_kernel(feat_ref, adj_hbm_ref, mask_ref,
                  e1w_ref, e1b_ref, e2w_ref, e2b_ref,
                  sgw_ref, sgb_ref, gdw_ref, gdb_ref,
                  p1w_ref, p1b_ref, p2w_ref, p2b_ref,
                  vw_ref, vb_ref, out_ref, adj_vmem, sems):
    n = adj_hbm_ref.shape[0]
    chunk = n // _N_CHUNKS

    copies = []
    for k in range(_N_CHUNKS):
        rows = pl.ds(k * chunk, chunk)
        cp = pltpu.make_async_copy(adj_hbm_ref.at[rows, :],
                                   adj_vmem.at[rows, :], sems.at[k])
        cp.start()
        copies.append(cp)

    x = jnp.maximum(
        jnp.dot(feat_ref[...], e1w_ref[...], preferred_element_type=_F32)
        + e1b_ref[...], 0.0)
    x = jnp.maximum(
        jnp.dot(x, e2w_ref[...], preferred_element_type=_F32)
        + e2b_ref[...], 0.0)

    deg = jnp.zeros((1, n), dtype=_F32)
    for k in range(_N_CHUNKS):
        copies[k].wait()
        deg = deg + jnp.sum(adj_vmem[pl.ds(k * chunk, chunk), :], axis=0,
                            keepdims=True)
    d_row = jnp.where(deg > 0.0, lax.rsqrt(deg), 0.0)
    d_col = d_row.reshape(n, 1)

    contract_rows = (((0,), (0,)), ((), ()))
    adj16 = adj_vmem[...].astype(jnp.bfloat16)
    h = x
    for _ in range(2):
        y = (d_col * h).astype(jnp.bfloat16)
        t = lax.dot_general(adj16, y, contract_rows,
                            preferred_element_type=_F32)
        h = d_col * t

    h = jnp.maximum(
        jnp.dot(h, sgw_ref[...], preferred_element_type=_F32)
        + sgb_ref[...], 0.0)
    x_graph = jnp.maximum(
        jnp.dot(h, gdw_ref[...], preferred_element_type=_F32)
        + gdb_ref[...], 0.0)

    f_graph = x_graph.shape[1]
    p = jnp.maximum(
        jnp.dot(x_graph, p1w_ref[:f_graph, :], preferred_element_type=_F32)
        + jnp.dot(x, p1w_ref[f_graph:, :], preferred_element_type=_F32)
        + p1b_ref[...], 0.0)
    p = jnp.maximum(
        jnp.dot(p, p2w_ref[...], preferred_element_type=_F32)
        + p2b_ref[...], 0.0)
    value = jnp.dot(p, vw_ref[...], preferred_element_type=_F32) + vb_ref[...]
    out_ref[...] = value * mask_ref[...]


def kernel(features, adjacency, mask, enc1_w, enc1_b, enc2_w, enc2_b,
           sg_w, sg_b, gd_w, gd_b, p1_w, p1_b, p2_w, p2_b, v_w, v_b):
    n = features.shape[0]
    args = (
        features, adjacency, mask.reshape(n, 1),
        enc1_w, enc1_b.reshape(1, -1), enc2_w, enc2_b.reshape(1, -1),
        sg_w, sg_b.reshape(1, -1), gd_w, gd_b.reshape(1, -1),
        p1_w, p1_b.reshape(1, -1),
        p2_w, p2_b.reshape(1, -1), v_w, v_b.reshape(1, -1),
    )
    in_specs = [pl.BlockSpec(memory_space=pl.ANY) if i == 1
                else pl.BlockSpec(memory_space=pltpu.MemorySpace.VMEM)
                for i in range(len(args))]
    return pl.pallas_call(
        _fused_kernel,
        out_shape=jax.ShapeDtypeStruct((n, 1), jnp.float32),
        in_specs=in_specs,
        scratch_shapes=[
            pltpu.VMEM((n, n), _F32),
            pltpu.SemaphoreType.DMA((_N_CHUNKS,)),
        ],
    )(*args)
